# baseline (speedup 1.0000x reference)
import functools

import jax
import jax.numpy as jnp
from jax import lax
from jax.experimental import pallas as pl
from jax.experimental.pallas import tpu as pltpu
from jax.experimental.pallas import tpu_sc as plsc

_LANES = 16
_SUBCORES = 16
_CORES = 2


def _ceil_to(v, m):
    return -(-v // m) * m


def _lin1_body(x_ref, w_ref, b_ref, h_ref, hn_ref):
    h = jnp.dot(x_ref[...], w_ref[...], preferred_element_type=jnp.float32)
    h = jnp.maximum(h + b_ref[...], 0.0)
    h_ref[...] = h
    nrm = jnp.sqrt(jnp.sum(h * h, axis=1, keepdims=True))
    hn_ref[...] = h / (nrm + 1e-12)


def _norm_body(p_ref, g_ref, gn_ref):
    g = p_ref[0] + p_ref[1]
    g_ref[...] = g
    nrm = jnp.sqrt(jnp.sum(g * g, axis=1, keepdims=True))
    gn_ref[...] = g / (nrm + 1e-12)


def _final_body(p_ref, w_ref, b_ref, o_ref):
    g = p_ref[0] + p_ref[1]
    logits = jnp.dot(g, w_ref[...], preferred_element_type=jnp.float32)
    logits = logits + b_ref[...]
    m = jnp.max(logits, axis=1, keepdims=True)
    z = logits - m
    o_ref[...] = z - jnp.log(jnp.sum(jnp.exp(z), axis=1, keepdims=True))


def _make_prop(npad, ep):
    ga = ep // (_SUBCORES * 128)
    nb = ga // 8
    nbh = -(-nb // _CORES)
    rpt = npad // _SUBCORES
    dz = npad // 8

    mesh = plsc.VectorSubcoreMesh(
        core_axis_name="c", subcore_axis_name="s",
        num_cores=_CORES, num_subcores=_SUBCORES)

    @functools.partial(
        pl.kernel,
        out_type=jax.ShapeDtypeStruct((_CORES * npad, _LANES), jnp.float32),
        mesh=mesh,
        scratch_types=[
            pltpu.VMEM_SHARED((npad,), jnp.float32),
            pltpu.VMEM_SHARED((npad, _LANES), jnp.float32),
            pltpu.VMEM((16, 128), jnp.int32),
            pltpu.VMEM((ga, 128), jnp.int32),
            pltpu.VMEM((ga, 128), jnp.float32),
            pltpu.VMEM((npad,), jnp.float32),
            pltpu.VMEM((128, _LANES), jnp.float32),
            pltpu.VMEM((128, _LANES), jnp.float32),
            pltpu.VMEM((128, _LANES), jnp.float32),
            pltpu.VMEM((128, _LANES), jnp.float32),
            pltpu.VMEM((dz,), jnp.float32),
            pltpu.VMEM((_LANES,), jnp.float32),
            pltpu.SemaphoreType.DMA,
            pltpu.SemaphoreType.DMA,
            pltpu.SemaphoreType.DMA,
            pltpu.SemaphoreType.DMA,
            pltpu.SemaphoreType.DMA,
            pltpu.SemaphoreType.DMA,
        ],
        compiler_params=pltpu.CompilerParams(
            needs_layout_passes=False, use_tc_tiling_on_sc=False),
    )
    def prop(h_hbm, hn_hbm, src_hbm, dst_hbm, beta_hbm, out_hbm,
             sh_den, sh_out,
             src16_v, dst_v, e_v, den_v,
             rows_a0, rows_b0, rows_a1, rows_b1,
             zbuf1, beta_v,
             sem_a0, sem_b0, sem_a1, sem_b1, sem_src, sem_sc):
        c = lax.axis_index("c")
        s = lax.axis_index("s")
        zero16 = jnp.zeros((_LANES,), jnp.float32)
        iota = lax.iota(jnp.int32, _LANES)
        r0 = pl.multiple_of(s * rpt, 8)
        pltpu.sync_copy(beta_hbm, beta_v)

        def _zrow(i, carry):
            rows_a0[i, :] = zero16
            return carry
        lax.fori_loop(0, 128, _zrow, 0)
        for i in range(rpt // 128):
            pltpu.sync_copy(rows_a0, sh_out.at[pl.ds(r0 + i * 128, 128)])
        rem = rpt % 128
        if rem:
            pltpu.sync_copy(rows_a0.at[pl.ds(0, rem)],
                            sh_out.at[pl.ds(r0 + (rpt // 128) * 128, rem)])

        def _zbuf(i, carry):
            zbuf1[pl.ds(i * _LANES, _LANES)] = zero16
            return carry
        lax.fori_loop(0, dz // _LANES, _zbuf, 0)

        @pl.when(s < 8)
        def _zden():
            pltpu.sync_copy(zbuf1, sh_den.at[pl.ds(pl.multiple_of(s * dz, 8), dz)])

        plsc.subcore_barrier()

        beta = beta_v[...]
        shift = jnp.abs(beta)
        pltpu.sync_copy(dst_hbm.at[pl.ds(pl.multiple_of(s * ga, 8), ga)], dst_v)

        def _src_refill(blk, half):
            return pltpu.make_async_copy(
                src_hbm.at[pl.ds(pl.multiple_of(s * ga + blk * 8, 8), 8)],
                src16_v.at[pl.ds(pl.multiple_of(half * 8, 8), 8)],
                sem_src)

        def _gath(table, idx_row, buf, sem):
            return pltpu.make_async_copy(table.at[idx_row], buf, sem)

        def _dot16(ra, rb, o):
            row16 = o * _LANES + iota
            acc = [zero16, zero16, zero16, zero16]
            for k in range(_LANES):
                kk = jnp.full((_LANES,), k, jnp.int32)
                ca = plsc.load_gather(ra, [row16, kk])
                cb = plsc.load_gather(rb, [row16, kk])
                acc[k % 4] = acc[k % 4] + ca * cb
            return (acc[0] + acc[1]) + (acc[2] + acc[3])

        def _compute_a(g, ra, rb):
            for o in range(128 // _LANES):
                c16 = _dot16(ra, rb, o)
                e16 = jnp.exp(beta * c16 - shift)
                e_v[g, pl.ds(o * _LANES, _LANES)] = e16

        def _sc_a_desc(g):
            return pltpu.make_async_copy(e_v.at[g], sh_den.at[dst_v.at[g]],
                                         sem_sc)

        def _scatter_a(g, last_g):
            @pl.when(last_g >= 0)
            def _():
                _sc_a_desc(last_g).wait()
            pltpu.async_copy(e_v.at[g], sh_den.at[dst_v.at[g]], sem_sc,
                             add=True)
            return g

        _src_refill(0, 0).start()

        def _block_a(blk, last_g):
            half = lax.rem(blk, 2)
            hrow = pl.multiple_of(half * 8, 8)
            _src_refill(blk, half).wait()

            @pl.when(blk + 1 < nb)
            def _():
                _src_refill(blk + 1, 1 - half).start()

            g0r = blk * 8
            _gath(hn_hbm, src16_v.at[hrow], rows_a0, sem_a0).start()
            _gath(hn_hbm, dst_v.at[g0r], rows_b0, sem_b0).start()

            def _pair(j, lg):
                ga_, gb_ = blk * 8 + 2 * j, blk * 8 + 2 * j + 1
                ja, jb = hrow + 2 * j, hrow + 2 * j + 1
                _gath(hn_hbm, src16_v.at[jb], rows_a1, sem_a1).start()
                _gath(hn_hbm, dst_v.at[gb_], rows_b1, sem_b1).start()
                _gath(hn_hbm, src16_v.at[ja], rows_a0, sem_a0).wait()
                _gath(hn_hbm, dst_v.at[ga_], rows_b0, sem_b0).wait()
                _compute_a(ga_, rows_a0, rows_b0)
                lg = _scatter_a(ga_, lg)

                @pl.when(2 * j + 2 < 8)
                def _():
                    _gath(hn_hbm, src16_v.at[ja + 2], rows_a0, sem_a0).start()
                    _gath(hn_hbm, dst_v.at[ga_ + 2], rows_b0, sem_b0).start()
                _gath(hn_hbm, src16_v.at[jb], rows_a1, sem_a1).wait()
                _gath(hn_hbm, dst_v.at[gb_], rows_b1, sem_b1).wait()
                _compute_a(gb_, rows_a1, rows_b1)
                lg = _scatter_a(gb_, lg)
                return lg
            return lax.fori_loop(0, 4, _pair, last_g)
        last_g = lax.fori_loop(0, nb, _block_a, jnp.int32(-1))
        _sc_a_desc(last_g).wait()
        plsc.subcore_barrier()

        pltpu.sync_copy(sh_den, den_v)

        def _compute_b(g, rh, rw):
            for o in range(128 // _LANES):
                row16 = o * _LANES + iota
                sl = pl.ds(o * _LANES, _LANES)
                d16 = plsc.load_gather(den_v, [dst_v[g, sl]])
                w16 = e_v[g, sl] / (d16 + 1e-16)
                for k in range(_LANES):
                    kk = jnp.full((_LANES,), k, jnp.int32)
                    col = plsc.load_gather(rh, [row16, kk])
                    plsc.store_scatter(rw, [row16, kk], col * w16)

        def _sc_b_desc(g, rw, sem):
            return pltpu.make_async_copy(rw, sh_out.at[dst_v.at[g]], sem)

        def _block_b(i, carry):
            lw0, lw1 = carry
            blk = i * _CORES + c

            def _do_block(carry2):
                lw0, lw1 = carry2
                pltpu.sync_copy(
                    src_hbm.at[pl.ds(pl.multiple_of(s * ga + blk * 8, 8), 8)],
                    src16_v.at[pl.ds(0, 8)])
                _gath(h_hbm, src16_v.at[0], rows_a0, sem_a0).start()

                def _pair(j, carry3):
                    lw0, lw1 = carry3
                    ga_, gb_ = blk * 8 + 2 * j, blk * 8 + 2 * j + 1
                    _gath(h_hbm, src16_v.at[2 * j + 1], rows_a1, sem_a1).start()
                    _gath(h_hbm, src16_v.at[2 * j], rows_a0, sem_a0).wait()
                    @pl.when(lw0 >= 0)
                    def _():
                        _sc_b_desc(lw0, rows_b0, sem_b0).wait()
                    _compute_b(ga_, rows_a0, rows_b0)
                    pltpu.async_copy(rows_b0, sh_out.at[dst_v.at[ga_]],
                                     sem_b0, add=True)
                    lw0 = ga_

                    @pl.when(2 * j + 2 < 8)
                    def _():
                        _gath(h_hbm, src16_v.at[2 * j + 2], rows_a0,
                              sem_a0).start()
                    _gath(h_hbm, src16_v.at[2 * j + 1], rows_a1, sem_a1).wait()
                    @pl.when(lw1 >= 0)
                    def _():
                        _sc_b_desc(lw1, rows_b1, sem_b1).wait()
                    _compute_b(gb_, rows_a1, rows_b1)
                    pltpu.async_copy(rows_b1, sh_out.at[dst_v.at[gb_]],
                                     sem_b1, add=True)
                    lw1 = gb_
                    return (lw0, lw1)
                return lax.fori_loop(0, 4, _pair, (lw0, lw1))
            return lax.cond(blk < nb, _do_block, lambda cc: cc, (lw0, lw1))
        lw0, lw1 = lax.fori_loop(0, nbh, _block_b,
                                 (jnp.int32(-1), jnp.int32(-1)))

        @pl.when(lw0 >= 0)
        def _():
            _sc_b_desc(lw0, rows_b0, sem_b0).wait()

        @pl.when(lw1 >= 0)
        def _():
            _sc_b_desc(lw1, rows_b1, sem_b1).wait()
        plsc.subcore_barrier()

        pltpu.sync_copy(sh_out.at[pl.ds(r0, rpt)],
                        out_hbm.at[pl.ds(pl.multiple_of(c * npad + r0, 8), rpt)])

    return prop


def kernel(x, edge_index, W1, b1, W2, b2, beta2):
    n, f_in = x.shape
    hid = W1.shape[1]
    nc = W2.shape[1]
    e = edge_index.shape[1]
    npad = _ceil_to(n, 128)
    ep = _ceil_to(e + n, _SUBCORES * 128 * 8)
    blk = npad // 16
    grid = npad // blk

    loops = jnp.arange(n, dtype=jnp.int32)
    padi = jnp.full((ep - e - n,), n, jnp.int32)
    src = jnp.concatenate([edge_index[0].astype(jnp.int32), loops, padi])
    dst = jnp.concatenate([edge_index[1].astype(jnp.int32), loops, padi])
    src2 = src.reshape(ep // 128, 128)
    dst2 = dst.reshape(ep // 128, 128)
    xp = jnp.pad(x, ((0, npad - n), (0, 0)))

    lin1 = pl.pallas_call(
        _lin1_body,
        grid=(grid,),
        in_specs=[
            pl.BlockSpec((blk, f_in), lambda i: (i, 0)),
            pl.BlockSpec((f_in, hid), lambda i: (0, 0)),
            pl.BlockSpec((1, hid), lambda i: (0, 0)),
        ],
        out_specs=[
            pl.BlockSpec((blk, hid), lambda i: (i, 0)),
            pl.BlockSpec((blk, hid), lambda i: (i, 0)),
        ],
        out_shape=[
            jax.ShapeDtypeStruct((npad, hid), jnp.float32),
            jax.ShapeDtypeStruct((npad, hid), jnp.float32),
        ],
    )
    h, hn = lin1(xp, W1, b1.reshape(1, hid))

    prop = _make_prop(npad, ep)
    ones16 = jnp.ones((_LANES,), jnp.float32)
    p1 = prop(h, hn, src2, dst2, ones16)

    norm = pl.pallas_call(
        _norm_body,
        grid=(grid,),
        in_specs=[pl.BlockSpec((2, blk, hid), lambda i: (0, i, 0))],
        out_specs=[
            pl.BlockSpec((blk, hid), lambda i: (i, 0)),
            pl.BlockSpec((blk, hid), lambda i: (i, 0)),
        ],
        out_shape=[
            jax.ShapeDtypeStruct((npad, hid), jnp.float32),
            jax.ShapeDtypeStruct((npad, hid), jnp.float32),
        ],
    )
    g1, g1n = norm(p1.reshape(2, npad, hid))

    beta16 = jnp.broadcast_to(beta2.astype(jnp.float32), (_LANES,))
    p2 = prop(g1, g1n, src2, dst2, beta16)

    final = pl.pallas_call(
        _final_body,
        grid=(grid,),
        in_specs=[
            pl.BlockSpec((2, blk, hid), lambda i: (0, i, 0)),
            pl.BlockSpec((hid, nc), lambda i: (0, 0)),
            pl.BlockSpec((1, nc), lambda i: (0, 0)),
        ],
        out_specs=pl.BlockSpec((blk, nc), lambda i: (i, 0)),
        out_shape=jax.ShapeDtypeStruct((npad, nc), jnp.float32),
    )
    out = final(p2.reshape(2, npad, hid), W2, b2.reshape(1, nc))
    return out[:n]


# speedup vs baseline: 3.5684x; 3.5684x over previous
import functools

import jax
import jax.numpy as jnp
from jax import lax
from jax.experimental import pallas as pl
from jax.experimental.pallas import tpu as pltpu
from jax.experimental.pallas import tpu_sc as plsc

_LANES = 16
_SUBCORES = 16
_CORES = 2


def _ceil_to(v, m):
    return -(-v // m) * m


def _lin1_body(x_ref, w_ref, b_ref, h_ref, hn_ref):
    h = jnp.dot(x_ref[...], w_ref[...], preferred_element_type=jnp.float32)
    h = jnp.maximum(h + b_ref[...], 0.0)
    h_ref[...] = h
    nrm = jnp.sqrt(jnp.sum(h * h, axis=1, keepdims=True))
    hn_ref[...] = h / (nrm + 1e-12)


def _norm_body(p_ref, g_ref, gn_ref):
    g = p_ref[0] + p_ref[1]
    g_ref[...] = g
    nrm = jnp.sqrt(jnp.sum(g * g, axis=1, keepdims=True))
    gn_ref[...] = g / (nrm + 1e-12)


def _final_body(p_ref, w_ref, b_ref, o_ref):
    g = p_ref[0] + p_ref[1]
    logits = jnp.dot(g, w_ref[...], preferred_element_type=jnp.float32)
    logits = logits + b_ref[...]
    m = jnp.max(logits, axis=1, keepdims=True)
    z = logits - m
    o_ref[...] = z - jnp.log(jnp.sum(jnp.exp(z), axis=1, keepdims=True))


def _make_prop(npad, ep):
    ga = ep // (_SUBCORES * 128)
    nb = ga // 8
    nbh = -(-nb // _CORES)
    rpt = npad // _SUBCORES
    dz = npad // 8

    mesh = plsc.VectorSubcoreMesh(
        core_axis_name="c", subcore_axis_name="s",
        num_cores=_CORES, num_subcores=_SUBCORES)

    @functools.partial(
        pl.kernel,
        out_type=jax.ShapeDtypeStruct((_CORES * npad, _LANES), jnp.float32),
        mesh=mesh,
        scratch_types=[
            pltpu.VMEM_SHARED((npad, _LANES), jnp.float32),
            pltpu.VMEM_SHARED((npad, _LANES), jnp.float32),
            pltpu.VMEM_SHARED((npad,), jnp.float32),
            pltpu.VMEM_SHARED((npad, _LANES), jnp.float32),
            pltpu.VMEM((16, 128), jnp.int32),
            pltpu.VMEM((ga, 128), jnp.int32),
            pltpu.VMEM((ga, 128), jnp.float32),
            pltpu.VMEM((npad,), jnp.float32),
            pltpu.VMEM((128, _LANES), jnp.float32),
            pltpu.VMEM((128, _LANES), jnp.float32),
            pltpu.VMEM((128, _LANES), jnp.float32),
            pltpu.VMEM((128, _LANES), jnp.float32),
            pltpu.VMEM((dz,), jnp.float32),
            pltpu.VMEM((_LANES,), jnp.float32),
            pltpu.VMEM((_LANES,), jnp.float32),
            pltpu.SemaphoreType.DMA,
            pltpu.SemaphoreType.DMA,
            pltpu.SemaphoreType.DMA,
            pltpu.SemaphoreType.DMA,
            pltpu.SemaphoreType.DMA,
            pltpu.SemaphoreType.DMA,
        ],
        compiler_params=pltpu.CompilerParams(
            needs_layout_passes=False, use_tc_tiling_on_sc=False),
    )
    def prop(h_hbm, hn_hbm, src_hbm, dst_hbm, beta_hbm, out_hbm,
             sh_h, sh_hn, sh_den, sh_out,
             src16_v, dst_v, e_v, den_v,
             rows_a0, rows_b0, rows_a1, rows_b1,
             zbuf1, beta_v, cstage,
             sem_a0, sem_b0, sem_a1, sem_b1, sem_src, sem_sc):
        c = lax.axis_index("c")
        s = lax.axis_index("s")
        zero16 = jnp.zeros((_LANES,), jnp.float32)
        iota = lax.iota(jnp.int32, _LANES)
        r0 = pl.multiple_of(s * rpt, 8)
        pltpu.sync_copy(h_hbm.at[pl.ds(r0, rpt)], sh_h.at[pl.ds(r0, rpt)])
        pltpu.sync_copy(hn_hbm.at[pl.ds(r0, rpt)], sh_hn.at[pl.ds(r0, rpt)])
        pltpu.sync_copy(beta_hbm, beta_v)

        def _zrow(i, carry):
            rows_a0[i, :] = zero16
            return carry
        lax.fori_loop(0, 128, _zrow, 0)
        for i in range(rpt // 128):
            pltpu.sync_copy(rows_a0, sh_out.at[pl.ds(r0 + i * 128, 128)])
        rem = rpt % 128
        if rem:
            pltpu.sync_copy(rows_a0.at[pl.ds(0, rem)],
                            sh_out.at[pl.ds(r0 + (rpt // 128) * 128, rem)])

        def _zbuf(i, carry):
            zbuf1[pl.ds(i * _LANES, _LANES)] = zero16
            return carry
        lax.fori_loop(0, dz // _LANES, _zbuf, 0)

        @pl.when(s < 8)
        def _zden():
            pltpu.sync_copy(zbuf1, sh_den.at[pl.ds(pl.multiple_of(s * dz, 8), dz)])

        plsc.subcore_barrier()

        beta = beta_v[...]
        shift = jnp.abs(beta)
        pltpu.sync_copy(dst_hbm.at[pl.ds(pl.multiple_of(s * ga, 8), ga)], dst_v)

        def _src_refill(blk, half):
            return pltpu.make_async_copy(
                src_hbm.at[pl.ds(pl.multiple_of(s * ga + blk * 8, 8), 8)],
                src16_v.at[pl.ds(pl.multiple_of(half * 8, 8), 8)],
                sem_src)

        def _gath(table, idx_row, buf, sem):
            return pltpu.make_async_copy(table.at[idx_row], buf, sem)

        def _compute_a(g, ra, rb):
            for o in range(128 // _LANES):
                c16 = zero16
                for j in range(_LANES):
                    ed = o * _LANES + j
                    p = ra[ed, :] * rb[ed, :]
                    tot = jnp.cumsum(p)[_LANES - 1]
                    c16 = jnp.where(iota == j, tot, c16)
                e16 = jnp.exp(beta * c16 - shift)
                e_v[g, pl.ds(o * _LANES, _LANES)] = e16

        def _sc_a_desc(g):
            return pltpu.make_async_copy(e_v.at[g], sh_den.at[dst_v.at[g]],
                                         sem_sc)

        def _scatter_a(g, last_g):
            @pl.when(last_g >= 0)
            def _():
                _sc_a_desc(last_g).wait()
            pltpu.async_copy(e_v.at[g], sh_den.at[dst_v.at[g]], sem_sc,
                             add=True)
            return g

        _src_refill(0, 0).start()

        def _block_a(blk, last_g):
            half = lax.rem(blk, 2)
            hrow = pl.multiple_of(half * 8, 8)
            _src_refill(blk, half).wait()

            @pl.when(blk + 1 < nb)
            def _():
                _src_refill(blk + 1, 1 - half).start()

            g0r = blk * 8
            _gath(sh_hn, src16_v.at[hrow], rows_a0, sem_a0).start()
            _gath(sh_hn, dst_v.at[g0r], rows_b0, sem_b0).start()

            def _pair(j, lg):
                ga_, gb_ = blk * 8 + 2 * j, blk * 8 + 2 * j + 1
                ja, jb = hrow + 2 * j, hrow + 2 * j + 1
                _gath(sh_hn, src16_v.at[jb], rows_a1, sem_a1).start()
                _gath(sh_hn, dst_v.at[gb_], rows_b1, sem_b1).start()
                _gath(sh_hn, src16_v.at[ja], rows_a0, sem_a0).wait()
                _gath(sh_hn, dst_v.at[ga_], rows_b0, sem_b0).wait()
                _compute_a(ga_, rows_a0, rows_b0)
                lg = _scatter_a(ga_, lg)

                @pl.when(2 * j + 2 < 8)
                def _():
                    _gath(sh_hn, src16_v.at[ja + 2], rows_a0, sem_a0).start()
                    _gath(sh_hn, dst_v.at[ga_ + 2], rows_b0, sem_b0).start()
                _gath(sh_hn, src16_v.at[jb], rows_a1, sem_a1).wait()
                _gath(sh_hn, dst_v.at[gb_], rows_b1, sem_b1).wait()
                _compute_a(gb_, rows_a1, rows_b1)
                lg = _scatter_a(gb_, lg)
                return lg
            return lax.fori_loop(0, 4, _pair, last_g)
        last_g = lax.fori_loop(0, nb, _block_a, jnp.int32(-1))
        _sc_a_desc(last_g).wait()
        plsc.subcore_barrier()

        pltpu.sync_copy(sh_den, den_v)

        def _compute_b(g, rh, rw):
            for o in range(128 // _LANES):
                sl = pl.ds(o * _LANES, _LANES)
                d16 = plsc.load_gather(den_v, [dst_v[g, sl]])
                w16 = e_v[g, sl] / (d16 + 1e-16)
                for j in range(_LANES):
                    ed = o * _LANES + j
                    rw[ed, :] = rh[ed, :] * w16[j]

        def _sc_b_desc(g, rw, sem):
            return pltpu.make_async_copy(rw, sh_out.at[dst_v.at[g]], sem)

        def _block_b(i, carry):
            lw0, lw1 = carry
            blk = i * _CORES + c

            def _do_block(carry2):
                lw0, lw1 = carry2
                pltpu.sync_copy(
                    src_hbm.at[pl.ds(pl.multiple_of(s * ga + blk * 8, 8), 8)],
                    src16_v.at[pl.ds(0, 8)])
                _gath(sh_h, src16_v.at[0], rows_a0, sem_a0).start()

                def _pair(j, carry3):
                    lw0, lw1 = carry3
                    ga_, gb_ = blk * 8 + 2 * j, blk * 8 + 2 * j + 1
                    _gath(sh_h, src16_v.at[2 * j + 1], rows_a1, sem_a1).start()
                    _gath(sh_h, src16_v.at[2 * j], rows_a0, sem_a0).wait()
                    @pl.when(lw0 >= 0)
                    def _():
                        _sc_b_desc(lw0, rows_b0, sem_b0).wait()
                    _compute_b(ga_, rows_a0, rows_b0)
                    pltpu.async_copy(rows_b0, sh_out.at[dst_v.at[ga_]],
                                     sem_b0, add=True)
                    lw0 = ga_

                    @pl.when(2 * j + 2 < 8)
                    def _():
                        _gath(sh_h, src16_v.at[2 * j + 2], rows_a0,
                              sem_a0).start()
                    _gath(sh_h, src16_v.at[2 * j + 1], rows_a1, sem_a1).wait()
                    @pl.when(lw1 >= 0)
                    def _():
                        _sc_b_desc(lw1, rows_b1, sem_b1).wait()
                    _compute_b(gb_, rows_a1, rows_b1)
                    pltpu.async_copy(rows_b1, sh_out.at[dst_v.at[gb_]],
                                     sem_b1, add=True)
                    lw1 = gb_
                    return (lw0, lw1)
                return lax.fori_loop(0, 4, _pair, (lw0, lw1))
            return lax.cond(blk < nb, _do_block, lambda cc: cc, (lw0, lw1))
        lw0, lw1 = lax.fori_loop(0, nbh, _block_b,
                                 (jnp.int32(-1), jnp.int32(-1)))

        @pl.when(lw0 >= 0)
        def _():
            _sc_b_desc(lw0, rows_b0, sem_b0).wait()

        @pl.when(lw1 >= 0)
        def _():
            _sc_b_desc(lw1, rows_b1, sem_b1).wait()
        plsc.subcore_barrier()

        pltpu.sync_copy(sh_out.at[pl.ds(r0, rpt)],
                        out_hbm.at[pl.ds(pl.multiple_of(c * npad + r0, 8), rpt)])

    return prop


def kernel(x, edge_index, W1, b1, W2, b2, beta2):
    n, f_in = x.shape
    hid = W1.shape[1]
    nc = W2.shape[1]
    e = edge_index.shape[1]
    npad = _ceil_to(n, 128)
    ep = _ceil_to(e + n, _SUBCORES * 128 * 8)
    blk = npad // 16
    grid = npad // blk

    loops = jnp.arange(n, dtype=jnp.int32)
    padi = jnp.full((ep - e - n,), n, jnp.int32)
    src = jnp.concatenate([edge_index[0].astype(jnp.int32), loops, padi])
    dst = jnp.concatenate([edge_index[1].astype(jnp.int32), loops, padi])
    src2 = src.reshape(ep // 128, 128)
    dst2 = dst.reshape(ep // 128, 128)
    xp = jnp.pad(x, ((0, npad - n), (0, 0)))

    lin1 = pl.pallas_call(
        _lin1_body,
        grid=(grid,),
        in_specs=[
            pl.BlockSpec((blk, f_in), lambda i: (i, 0)),
            pl.BlockSpec((f_in, hid), lambda i: (0, 0)),
            pl.BlockSpec((1, hid), lambda i: (0, 0)),
        ],
        out_specs=[
            pl.BlockSpec((blk, hid), lambda i: (i, 0)),
            pl.BlockSpec((blk, hid), lambda i: (i, 0)),
        ],
        out_shape=[
            jax.ShapeDtypeStruct((npad, hid), jnp.float32),
            jax.ShapeDtypeStruct((npad, hid), jnp.float32),
        ],
    )
    h, hn = lin1(xp, W1, b1.reshape(1, hid))

    prop = _make_prop(npad, ep)
    ones16 = jnp.ones((_LANES,), jnp.float32)
    p1 = prop(h, hn, src2, dst2, ones16)

    norm = pl.pallas_call(
        _norm_body,
        grid=(grid,),
        in_specs=[pl.BlockSpec((2, blk, hid), lambda i: (0, i, 0))],
        out_specs=[
            pl.BlockSpec((blk, hid), lambda i: (i, 0)),
            pl.BlockSpec((blk, hid), lambda i: (i, 0)),
        ],
        out_shape=[
            jax.ShapeDtypeStruct((npad, hid), jnp.float32),
            jax.ShapeDtypeStruct((npad, hid), jnp.float32),
        ],
    )
    g1, g1n = norm(p1.reshape(2, npad, hid))

    beta16 = jnp.broadcast_to(beta2.astype(jnp.float32), (_LANES,))
    p2 = prop(g1, g1n, src2, dst2, beta16)

    final = pl.pallas_call(
        _final_body,
        grid=(grid,),
        in_specs=[
            pl.BlockSpec((2, blk, hid), lambda i: (0, i, 0)),
            pl.BlockSpec((hid, nc), lambda i: (0, 0)),
            pl.BlockSpec((1, nc), lambda i: (0, 0)),
        ],
        out_specs=pl.BlockSpec((blk, nc), lambda i: (i, 0)),
        out_shape=jax.ShapeDtypeStruct((npad, nc), jnp.float32),
    )
    out = final(p2.reshape(2, npad, hid), W2, b2.reshape(1, nc))
    return out[:n]


# speedup vs baseline: 3.9001x; 1.0929x over previous
import functools

import jax
import jax.numpy as jnp
from jax import lax
from jax.experimental import pallas as pl
from jax.experimental.pallas import tpu as pltpu
from jax.experimental.pallas import tpu_sc as plsc

_LANES = 16
_SUBCORES = 16
_CORES = 2


def _ceil_to(v, m):
    return -(-v // m) * m


def _lin1_body(x_ref, w_ref, b_ref, h_ref, hn_ref):
    h = jnp.dot(x_ref[...], w_ref[...], preferred_element_type=jnp.float32)
    h = jnp.maximum(h + b_ref[...], 0.0)
    h_ref[...] = h
    nrm = jnp.sqrt(jnp.sum(h * h, axis=1, keepdims=True))
    hn_ref[...] = h / (nrm + 1e-12)


def _norm_body(p_ref, g_ref, gn_ref):
    g = p_ref[0] + p_ref[1]
    g_ref[...] = g
    nrm = jnp.sqrt(jnp.sum(g * g, axis=1, keepdims=True))
    gn_ref[...] = g / (nrm + 1e-12)


def _final_body(p_ref, w_ref, b_ref, o_ref):
    g = p_ref[0] + p_ref[1]
    logits = jnp.dot(g, w_ref[...], preferred_element_type=jnp.float32)
    logits = logits + b_ref[...]
    m = jnp.max(logits, axis=1, keepdims=True)
    z = logits - m
    o_ref[...] = z - jnp.log(jnp.sum(jnp.exp(z), axis=1, keepdims=True))


def _make_prop_a(npad, ep):
    ga = ep // (_SUBCORES * 128)
    nb = ga // 8
    nbh = -(-nb // _CORES)
    rpt = npad // _SUBCORES
    dz = npad // 8

    mesh = plsc.VectorSubcoreMesh(
        core_axis_name="c", subcore_axis_name="s",
        num_cores=_CORES, num_subcores=_SUBCORES)

    @functools.partial(
        pl.kernel,
        out_type=[
            jax.ShapeDtypeStruct((_CORES * npad,), jnp.float32),
            jax.ShapeDtypeStruct((ep // 128, 128), jnp.float32),
        ],
        mesh=mesh,
        scratch_types=[
            pltpu.VMEM_SHARED((npad, _LANES), jnp.float32),
            pltpu.VMEM_SHARED((npad,), jnp.float32),
            pltpu.VMEM((8, 128), jnp.int32),
            pltpu.VMEM((ga, 128), jnp.int32),
            pltpu.VMEM((ga, 128), jnp.float32),
            pltpu.VMEM((128, _LANES), jnp.float32),
            pltpu.VMEM((128, _LANES), jnp.float32),
            pltpu.VMEM((128, _LANES), jnp.float32),
            pltpu.VMEM((128, _LANES), jnp.float32),
            pltpu.VMEM((dz,), jnp.float32),
            pltpu.VMEM((_LANES,), jnp.float32),
            pltpu.SemaphoreType.DMA,
            pltpu.SemaphoreType.DMA,
            pltpu.SemaphoreType.DMA,
            pltpu.SemaphoreType.DMA,
            pltpu.SemaphoreType.DMA,
        ],
        compiler_params=pltpu.CompilerParams(
            needs_layout_passes=False, use_tc_tiling_on_sc=False),
    )
    def prop_a(hn_hbm, src_hbm, dst_hbm, beta_hbm, den_hbm, e_hbm,
               sh_hn, sh_den,
               src8_v, dst_v, e_v, rows_a0, rows_b0, rows_a1, rows_b1,
               zbuf1, beta_v,
               sem_a0, sem_b0, sem_a1, sem_b1, sem_sc):
        c = lax.axis_index("c")
        s = lax.axis_index("s")
        zero16 = jnp.zeros((_LANES,), jnp.float32)
        iota = lax.iota(jnp.int32, _LANES)

        r0 = pl.multiple_of(s * rpt, 8)
        pltpu.sync_copy(hn_hbm.at[pl.ds(r0, rpt)], sh_hn.at[pl.ds(r0, rpt)])
        pltpu.sync_copy(beta_hbm, beta_v)

        def _zbuf(i, carry):
            zbuf1[pl.ds(i * _LANES, _LANES)] = zero16
            return carry
        lax.fori_loop(0, dz // _LANES, _zbuf, 0)

        @pl.when(s < 8)
        def _zden():
            pltpu.sync_copy(zbuf1, sh_den.at[pl.ds(pl.multiple_of(s * dz, 8), dz)])

        plsc.subcore_barrier()

        beta = beta_v[...]
        shift = jnp.abs(beta)
        pltpu.sync_copy(dst_hbm.at[pl.ds(pl.multiple_of(s * ga, 8), ga)], dst_v)

        def _gath(idx_row, buf, sem):
            return pltpu.make_async_copy(sh_hn.at[idx_row], buf, sem)

        def _compute_a(g, ra, rb):
            for o in range(128 // _LANES):
                c16 = zero16
                for j in range(_LANES):
                    ed = o * _LANES + j
                    p = ra[ed, :] * rb[ed, :]
                    tot = jnp.cumsum(p)[_LANES - 1]
                    c16 = jnp.where(iota == j, tot, c16)
                e16 = jnp.exp(beta * c16 - shift)
                e_v[g, pl.ds(o * _LANES, _LANES)] = e16

        def _sc_a_desc(g):
            return pltpu.make_async_copy(e_v.at[g], sh_den.at[dst_v.at[g]],
                                         sem_sc)

        def _scatter_a(g, last_g):
            @pl.when(last_g >= 0)
            def _():
                _sc_a_desc(last_g).wait()
            pltpu.async_copy(e_v.at[g], sh_den.at[dst_v.at[g]], sem_sc,
                             add=True)
            return g

        def _block_a(i, last_g):
            blk = i * _CORES + c

            def _do_block(lg):
                pltpu.sync_copy(
                    src_hbm.at[pl.ds(pl.multiple_of(s * ga + blk * 8, 8), 8)],
                    src8_v)
                g0r = blk * 8
                _gath(src8_v.at[0], rows_a0, sem_a0).start()
                _gath(dst_v.at[g0r], rows_b0, sem_b0).start()

                def _pair(j, lg2):
                    ga_, gb_ = blk * 8 + 2 * j, blk * 8 + 2 * j + 1
                    _gath(src8_v.at[2 * j + 1], rows_a1, sem_a1).start()
                    _gath(dst_v.at[gb_], rows_b1, sem_b1).start()
                    _gath(src8_v.at[2 * j], rows_a0, sem_a0).wait()
                    _gath(dst_v.at[ga_], rows_b0, sem_b0).wait()
                    _compute_a(ga_, rows_a0, rows_b0)
                    lg2 = _scatter_a(ga_, lg2)

                    @pl.when(2 * j + 2 < 8)
                    def _():
                        _gath(src8_v.at[2 * j + 2], rows_a0, sem_a0).start()
                        _gath(dst_v.at[ga_ + 2], rows_b0, sem_b0).start()
                    _gath(src8_v.at[2 * j + 1], rows_a1, sem_a1).wait()
                    _gath(dst_v.at[gb_], rows_b1, sem_b1).wait()
                    _compute_a(gb_, rows_a1, rows_b1)
                    lg2 = _scatter_a(gb_, lg2)
                    return lg2
                lg = lax.fori_loop(0, 4, _pair, lg)
                pltpu.sync_copy(
                    e_v.at[pl.ds(pl.multiple_of(blk * 8, 8), 8)],
                    e_hbm.at[pl.ds(pl.multiple_of(s * ga + blk * 8, 8), 8)])
                return lg
            return lax.cond(blk < nb, _do_block, lambda lg: lg, last_g)
        last_g = lax.fori_loop(0, nbh, _block_a, jnp.int32(-1))

        @pl.when(last_g >= 0)
        def _():
            _sc_a_desc(last_g).wait()
        plsc.subcore_barrier()

        @pl.when(s == 0)
        def _copy_den():
            pltpu.sync_copy(sh_den,
                            den_hbm.at[pl.ds(pl.multiple_of(c * npad, 8), npad)])

    return prop_a


def _make_prop_b(npad, ep):
    ga = ep // (_SUBCORES * 128)
    nb = ga // 8
    nbh = -(-nb // _CORES)
    rpt = npad // _SUBCORES

    mesh = plsc.VectorSubcoreMesh(
        core_axis_name="c", subcore_axis_name="s",
        num_cores=_CORES, num_subcores=_SUBCORES)

    @functools.partial(
        pl.kernel,
        out_type=jax.ShapeDtypeStruct((_CORES * npad, _LANES), jnp.float32),
        mesh=mesh,
        scratch_types=[
            pltpu.VMEM_SHARED((npad, _LANES), jnp.float32),
            pltpu.VMEM_SHARED((npad, _LANES), jnp.float32),
            pltpu.VMEM((8, 128), jnp.int32),
            pltpu.VMEM((ga, 128), jnp.int32),
            pltpu.VMEM((ga, 128), jnp.float32),
            pltpu.VMEM((npad,), jnp.float32),
            pltpu.VMEM((npad,), jnp.float32),
            pltpu.VMEM((128, _LANES), jnp.float32),
            pltpu.VMEM((128, _LANES), jnp.float32),
            pltpu.VMEM((128, _LANES), jnp.float32),
            pltpu.VMEM((128, _LANES), jnp.float32),
            pltpu.SemaphoreType.DMA,
            pltpu.SemaphoreType.DMA,
            pltpu.SemaphoreType.DMA,
            pltpu.SemaphoreType.DMA,
        ],
        compiler_params=pltpu.CompilerParams(
            needs_layout_passes=False, use_tc_tiling_on_sc=False),
    )
    def prop_b(h_hbm, src_hbm, dst_hbm, den_hbm, e_hbm, out_hbm,
               sh_h, sh_out,
               src8_v, dst_v, e_v, den_v, den_b,
               rows_a0, rows_b0, rows_a1, rows_b1,
               sem_a0, sem_b0, sem_a1, sem_b1):
        c = lax.axis_index("c")
        s = lax.axis_index("s")
        zero16 = jnp.zeros((_LANES,), jnp.float32)

        r0 = pl.multiple_of(s * rpt, 8)
        pltpu.sync_copy(h_hbm.at[pl.ds(r0, rpt)], sh_h.at[pl.ds(r0, rpt)])

        def _zrow(i, carry):
            rows_a0[i, :] = zero16
            return carry
        lax.fori_loop(0, 128, _zrow, 0)
        for i in range(rpt // 128):
            pltpu.sync_copy(rows_a0, sh_out.at[pl.ds(r0 + i * 128, 128)])
        rem = rpt % 128
        if rem:
            pltpu.sync_copy(rows_a0.at[pl.ds(0, rem)],
                            sh_out.at[pl.ds(r0 + (rpt // 128) * 128, rem)])

        pltpu.sync_copy(den_hbm.at[pl.ds(0, npad)], den_v)
        pltpu.sync_copy(den_hbm.at[pl.ds(npad, npad)], den_b)

        def _dadd(i, carry):
            sl = pl.ds(i * _LANES, _LANES)
            den_v[sl] = den_v[sl] + den_b[sl]
            return carry
        lax.fori_loop(0, npad // _LANES, _dadd, 0)

        pltpu.sync_copy(dst_hbm.at[pl.ds(pl.multiple_of(s * ga, 8), ga)], dst_v)
        pltpu.sync_copy(e_hbm.at[pl.ds(pl.multiple_of(s * ga, 8), ga)], e_v)
        plsc.subcore_barrier()

        def _gath(idx_row, buf, sem):
            return pltpu.make_async_copy(sh_h.at[idx_row], buf, sem)

        def _compute_b(g, rh, rw):
            for o in range(128 // _LANES):
                sl = pl.ds(o * _LANES, _LANES)
                d16 = plsc.load_gather(den_v, [dst_v[g, sl]])
                w16 = e_v[g, sl] / (d16 + 1e-16)
                for j in range(_LANES):
                    ed = o * _LANES + j
                    rw[ed, :] = rh[ed, :] * w16[j]

        def _sc_b_desc(g, rw, sem):
            return pltpu.make_async_copy(rw, sh_out.at[dst_v.at[g]], sem)

        def _block_b(i, carry):
            lw0, lw1 = carry
            blk = i * _CORES + c

            def _do_block(carry2):
                lw0, lw1 = carry2
                pltpu.sync_copy(
                    src_hbm.at[pl.ds(pl.multiple_of(s * ga + blk * 8, 8), 8)],
                    src8_v)
                _gath(src8_v.at[0], rows_a0, sem_a0).start()

                def _pair(j, carry3):
                    lw0, lw1 = carry3
                    ga_, gb_ = blk * 8 + 2 * j, blk * 8 + 2 * j + 1
                    _gath(src8_v.at[2 * j + 1], rows_a1, sem_a1).start()
                    _gath(src8_v.at[2 * j], rows_a0, sem_a0).wait()
                    @pl.when(lw0 >= 0)
                    def _():
                        _sc_b_desc(lw0, rows_b0, sem_b0).wait()
                    _compute_b(ga_, rows_a0, rows_b0)
                    pltpu.async_copy(rows_b0, sh_out.at[dst_v.at[ga_]],
                                     sem_b0, add=True)
                    lw0 = ga_

                    @pl.when(2 * j + 2 < 8)
                    def _():
                        _gath(src8_v.at[2 * j + 2], rows_a0, sem_a0).start()
                    _gath(src8_v.at[2 * j + 1], rows_a1, sem_a1).wait()
                    @pl.when(lw1 >= 0)
                    def _():
                        _sc_b_desc(lw1, rows_b1, sem_b1).wait()
                    _compute_b(gb_, rows_a1, rows_b1)
                    pltpu.async_copy(rows_b1, sh_out.at[dst_v.at[gb_]],
                                     sem_b1, add=True)
                    lw1 = gb_
                    return (lw0, lw1)
                return lax.fori_loop(0, 4, _pair, (lw0, lw1))
            return lax.cond(blk < nb, _do_block, lambda cc: cc, (lw0, lw1))
        lw0, lw1 = lax.fori_loop(0, nbh, _block_b,
                                 (jnp.int32(-1), jnp.int32(-1)))

        @pl.when(lw0 >= 0)
        def _():
            _sc_b_desc(lw0, rows_b0, sem_b0).wait()

        @pl.when(lw1 >= 0)
        def _():
            _sc_b_desc(lw1, rows_b1, sem_b1).wait()
        plsc.subcore_barrier()

        pltpu.sync_copy(sh_out.at[pl.ds(r0, rpt)],
                        out_hbm.at[pl.ds(pl.multiple_of(c * npad + r0, 8), rpt)])

    return prop_b


def kernel(x, edge_index, W1, b1, W2, b2, beta2):
    n, f_in = x.shape
    hid = W1.shape[1]
    nc = W2.shape[1]
    e = edge_index.shape[1]
    npad = _ceil_to(n, 128)
    ep = _ceil_to(e + n, _SUBCORES * 128 * 8)
    blk = npad // 16
    grid = npad // blk

    loops = jnp.arange(n, dtype=jnp.int32)
    padi = jnp.full((ep - e - n,), n, jnp.int32)
    src = jnp.concatenate([edge_index[0].astype(jnp.int32), loops, padi])
    dst = jnp.concatenate([edge_index[1].astype(jnp.int32), loops, padi])
    src2 = src.reshape(ep // 128, 128)
    dst2 = dst.reshape(ep // 128, 128)
    xp = jnp.pad(x, ((0, npad - n), (0, 0)))

    lin1 = pl.pallas_call(
        _lin1_body,
        grid=(grid,),
        in_specs=[
            pl.BlockSpec((blk, f_in), lambda i: (i, 0)),
            pl.BlockSpec((f_in, hid), lambda i: (0, 0)),
            pl.BlockSpec((1, hid), lambda i: (0, 0)),
        ],
        out_specs=[
            pl.BlockSpec((blk, hid), lambda i: (i, 0)),
            pl.BlockSpec((blk, hid), lambda i: (i, 0)),
        ],
        out_shape=[
            jax.ShapeDtypeStruct((npad, hid), jnp.float32),
            jax.ShapeDtypeStruct((npad, hid), jnp.float32),
        ],
    )
    h, hn = lin1(xp, W1, b1.reshape(1, hid))

    prop_a = _make_prop_a(npad, ep)
    prop_b = _make_prop_b(npad, ep)
    ones16 = jnp.ones((_LANES,), jnp.float32)
    den1, e1 = prop_a(hn, src2, dst2, ones16)
    p1 = prop_b(h, src2, dst2, den1, e1)

    norm = pl.pallas_call(
        _norm_body,
        grid=(grid,),
        in_specs=[pl.BlockSpec((2, blk, hid), lambda i: (0, i, 0))],
        out_specs=[
            pl.BlockSpec((blk, hid), lambda i: (i, 0)),
            pl.BlockSpec((blk, hid), lambda i: (i, 0)),
        ],
        out_shape=[
            jax.ShapeDtypeStruct((npad, hid), jnp.float32),
            jax.ShapeDtypeStruct((npad, hid), jnp.float32),
        ],
    )
    g1, g1n = norm(p1.reshape(2, npad, hid))

    beta16 = jnp.broadcast_to(beta2.astype(jnp.float32), (_LANES,))
    den2, e2 = prop_a(g1n, src2, dst2, beta16)
    p2 = prop_b(g1, src2, dst2, den2, e2)

    final = pl.pallas_call(
        _final_body,
        grid=(grid,),
        in_specs=[
            pl.BlockSpec((2, blk, hid), lambda i: (0, i, 0)),
            pl.BlockSpec((hid, nc), lambda i: (0, 0)),
            pl.BlockSpec((1, nc), lambda i: (0, 0)),
        ],
        out_specs=pl.BlockSpec((blk, nc), lambda i: (i, 0)),
        out_shape=jax.ShapeDtypeStruct((npad, nc), jnp.float32),
    )
    out = final(p2.reshape(2, npad, hid), W2, b2.reshape(1, nc))
    return out[:n]
